# Initial kernel scaffold; baseline (speedup 1.0000x reference)
import jax
import jax.numpy as jnp
from jax.experimental import pallas as pl


def kernel(input, table, W1, b1, W2, b2):
    raise NotImplementedError("write your pallas kernel here")


# speedup vs baseline: 13.2151x; 13.2151x over previous
import functools

import jax
import jax.numpy as jnp
from jax import lax
from jax.experimental import pallas as pl
from jax.experimental.pallas import tpu as pltpu
from jax.experimental.pallas import tpu_sc as plsc

NC, NS = 2, 16
NW = NC * NS
CHUNK = 128


@functools.partial(jax.jit, static_argnums=(2, 3))
def _sc_gather(idx3, table, total, d):
    nchunk = idx3.shape[1]
    mesh = plsc.VectorSubcoreMesh(
        core_axis_name="c", subcore_axis_name="s", num_cores=NC, num_subcores=NS
    )

    def body(idx_hbm, table_hbm, out_hbm, idx_v, rows_v, sem):
        wid = lax.axis_index("s") * NC + lax.axis_index("c")
        pltpu.sync_copy(idx_hbm.at[wid], idx_v)
        base = wid * (nchunk * CHUNK)

        def step(j, carry):
            pltpu.async_copy(table_hbm.at[idx_v.at[j]], rows_v, sem).wait()
            pltpu.sync_copy(rows_v, out_hbm.at[pl.ds(base + j * CHUNK, CHUNK)])
            return carry

        lax.fori_loop(0, nchunk, step, 0)

    run = pl.kernel(
        body,
        out_type=jax.ShapeDtypeStruct((total, d), jnp.float32),
        mesh=mesh,
        scratch_types=[
            pltpu.VMEM((nchunk, CHUNK), jnp.int32),
            pltpu.VMEM((CHUNK, d), jnp.float32),
            pltpu.SemaphoreType.DMA,
        ],
        compiler_params=pltpu.CompilerParams(use_tc_tiling_on_sc=False),
    )
    return run(idx3, table)


def _mlp_body(emb_ref, w1_ref, b1_ref, w2_ref, b2_ref, out_ref):
    h = jnp.dot(emb_ref[...], w1_ref[...], preferred_element_type=jnp.float32)
    h = jnp.maximum(h + b1_ref[...], 0.0)
    o = jnp.dot(h, w2_ref[...], preferred_element_type=jnp.float32) + b2_ref[...]
    out_ref[...] = jax.nn.sigmoid(o)


def _mlp(emb, W1, b1, W2, b2, block_m=1024):
    B, d_in = emb.shape
    d_hid = W1.shape[1]
    d_out = W2.shape[1]
    grid = (B // block_m,)
    return pl.pallas_call(
        _mlp_body,
        grid=grid,
        in_specs=[
            pl.BlockSpec((block_m, d_in), lambda i: (i, 0)),
            pl.BlockSpec((d_in, d_hid), lambda i: (0, 0)),
            pl.BlockSpec((1, d_hid), lambda i: (0, 0)),
            pl.BlockSpec((d_hid, d_out), lambda i: (0, 0)),
            pl.BlockSpec((1, d_out), lambda i: (0, 0)),
        ],
        out_specs=pl.BlockSpec((block_m, d_out), lambda i: (i, 0)),
        out_shape=jax.ShapeDtypeStruct((B, d_out), jnp.float32),
    )(emb, W1, b1.reshape(1, d_hid), W2, b2.reshape(1, d_out))


def kernel(input, table, W1, b1, W2, b2):
    B, K = input.shape
    d = table.shape[1]
    total = B * K
    per_w = total // NW
    nchunk = per_w // CHUNK
    idx3 = input.reshape(NW, nchunk, CHUNK)
    rows = _sc_gather(idx3, table, total, d)
    emb = rows.reshape(B, K * d)
    return _mlp(emb, W1, b1, W2, b2)


# speedup vs baseline: 14.0235x; 1.0612x over previous
import functools

import jax
import jax.numpy as jnp
from jax import lax
from jax.experimental import pallas as pl
from jax.experimental.pallas import tpu as pltpu
from jax.experimental.pallas import tpu_sc as plsc

NC, NS = 2, 16
NW = NC * NS
CHUNK = 128


def _tpose_body(tt_ref, out_ref):
    x = tt_ref[...]
    y = x.T
    z = jnp.zeros((y.shape[0], 128 - y.shape[1]), jnp.float32)
    out_ref[...] = jnp.concatenate([y, z], axis=1)


def _tpose(tableT, bn=2048):
    d, v = tableT.shape
    grid = (pl.cdiv(v, bn),)
    return pl.pallas_call(
        _tpose_body,
        grid=grid,
        in_specs=[pl.BlockSpec((d, bn), lambda i: (0, i))],
        out_specs=pl.BlockSpec((bn, 128), lambda i: (i, 0)),
        out_shape=jax.ShapeDtypeStruct((v, 128), jnp.float32),
    )(tableT)


def _sc_gather(idx3, table_lin, total, d):
    nchunk = idx3.shape[1]
    mesh = plsc.VectorSubcoreMesh(
        core_axis_name="c", subcore_axis_name="s", num_cores=NC, num_subcores=NS
    )

    def body(idx_hbm, table_hbm, out_hbm, idx_v, rows_v, sem):
        wid = lax.axis_index("s") * NC + lax.axis_index("c")
        pltpu.sync_copy(idx_hbm.at[wid], idx_v)
        base = wid * (nchunk * CHUNK)

        def step(j, carry):
            pltpu.async_copy(table_hbm.at[idx_v.at[j]], rows_v, sem).wait()
            pltpu.sync_copy(rows_v, out_hbm.at[pl.ds(base + j * CHUNK, CHUNK)])
            return carry

        lax.fori_loop(0, nchunk, step, 0)

    run = pl.kernel(
        body,
        out_type=jax.ShapeDtypeStruct((total, d), jnp.float32),
        mesh=mesh,
        scratch_types=[
            pltpu.VMEM((nchunk, CHUNK), jnp.int32),
            pltpu.VMEM((CHUNK, d), jnp.float32),
            pltpu.SemaphoreType.DMA,
        ],
        compiler_params=pltpu.CompilerParams(use_tc_tiling_on_sc=False),
    )
    return run(idx3, table_lin)


def _mlp_body(emb_ref, w1_ref, b1_ref, w2_ref, b2_ref, out_ref):
    h = jnp.dot(emb_ref[...], w1_ref[...], preferred_element_type=jnp.float32)
    h = jnp.maximum(h + b1_ref[...], 0.0)
    o = jnp.dot(h, w2_ref[...], preferred_element_type=jnp.float32) + b2_ref[...]
    out_ref[...] = jax.nn.sigmoid(o)


def _mlp(emb, W1, b1, W2, b2, block_m=1024):
    B, d_in = emb.shape
    d_hid = W1.shape[1]
    d_out = W2.shape[1]
    grid = (B // block_m,)
    return pl.pallas_call(
        _mlp_body,
        grid=grid,
        in_specs=[
            pl.BlockSpec((block_m, d_in), lambda i: (i, 0)),
            pl.BlockSpec((d_in, d_hid), lambda i: (0, 0)),
            pl.BlockSpec((1, d_hid), lambda i: (0, 0)),
            pl.BlockSpec((d_hid, d_out), lambda i: (0, 0)),
            pl.BlockSpec((1, d_out), lambda i: (0, 0)),
        ],
        out_specs=pl.BlockSpec((block_m, d_out), lambda i: (i, 0)),
        out_shape=jax.ShapeDtypeStruct((B, d_out), jnp.float32),
    )(emb, W1, b1.reshape(1, d_hid), W2, b2.reshape(1, d_out))


def kernel(input, table, W1, b1, W2, b2):
    B, K = input.shape
    V, d = table.shape
    total = B * K
    per_w = total // NW
    nchunk = per_w // CHUNK
    idx3 = (input * 4).reshape(NW, nchunk, CHUNK)
    table_pad = _tpose(table.T)
    table_lin = table_pad.reshape(4 * V, d)
    rows = _sc_gather(idx3, table_lin, total, d)
    emb = rows.reshape(B, K * d)
    return _mlp(emb, W1, b1, W2, b2)


# speedup vs baseline: 21.6524x; 1.5440x over previous
import functools

import jax
import jax.numpy as jnp
from jax import lax
from jax.experimental import pallas as pl
from jax.experimental.pallas import tpu as pltpu
from jax.experimental.pallas import tpu_sc as plsc

NC, NS = 2, 16
NW = NC * NS
CHUNK = 128


def _tpose_body(tt_ref, out_ref):
    x = tt_ref[...]
    y = x.T
    z = jnp.zeros((y.shape[0], 128 - y.shape[1]), jnp.float32)
    out_ref[...] = jnp.concatenate([y, z], axis=1)


def _tpose(tableT, bn=8192):
    d, v = tableT.shape
    grid = (pl.cdiv(v, bn),)
    return pl.pallas_call(
        _tpose_body,
        grid=grid,
        in_specs=[pl.BlockSpec((d, bn), lambda i: (0, i))],
        out_specs=pl.BlockSpec((bn, 128), lambda i: (i, 0)),
        out_shape=jax.ShapeDtypeStruct((v, 128), jnp.float32),
        compiler_params=pltpu.CompilerParams(
            dimension_semantics=("arbitrary",)
        ),
    )(tableT)


def _sc_gather(idx3, table_lin, total, d):
    nchunk = idx3.shape[1]
    mesh = plsc.VectorSubcoreMesh(
        core_axis_name="c", subcore_axis_name="s", num_cores=NC, num_subcores=NS
    )

    nbuf = 4
    assert nchunk % nbuf == 0

    def body(idx_hbm, table_hbm, out_hbm, idx_v, rows_v, sems):
        wid = lax.axis_index("s") * NC + lax.axis_index("c")
        pltpu.sync_copy(idx_hbm.at[wid], idx_v)
        base = wid * (nchunk * CHUNK)

        def step(j2, carry):
            j0 = j2 * nbuf
            cps = [
                pltpu.async_copy(
                    table_hbm.at[idx_v.at[j0 + b]], rows_v.at[b], sems.at[b]
                )
                for b in range(nbuf)
            ]
            for b in range(nbuf):
                cps[b].wait()
                pltpu.sync_copy(
                    rows_v.at[b],
                    out_hbm.at[pl.ds(base + (j0 + b) * CHUNK, CHUNK)],
                )
            return carry

        lax.fori_loop(0, nchunk // nbuf, step, 0)

    run = pl.kernel(
        body,
        out_type=jax.ShapeDtypeStruct((total, d), jnp.float32),
        mesh=mesh,
        scratch_types=[
            pltpu.VMEM((nchunk, CHUNK), jnp.int32),
            pltpu.VMEM((nbuf, CHUNK, d), jnp.float32),
            pltpu.SemaphoreType.DMA((nbuf,)),
        ],
        compiler_params=pltpu.CompilerParams(use_tc_tiling_on_sc=False),
    )
    return run(idx3, table_lin)


def _mlp_body(emb_ref, w1_ref, b1_ref, w2_ref, b2_ref, out_ref):
    h = jnp.dot(emb_ref[...], w1_ref[...], preferred_element_type=jnp.float32)
    h = jnp.maximum(h + b1_ref[...], 0.0)
    o = jnp.dot(h, w2_ref[...], preferred_element_type=jnp.float32) + b2_ref[...]
    out_ref[...] = jax.nn.sigmoid(o)


def _mlp(emb, W1, b1, W2, b2, block_m=1024):
    B, d_in = emb.shape
    d_hid = W1.shape[1]
    d_out = W2.shape[1]
    grid = (B // block_m,)
    return pl.pallas_call(
        _mlp_body,
        grid=grid,
        in_specs=[
            pl.BlockSpec((block_m, d_in), lambda i: (i, 0)),
            pl.BlockSpec((d_in, d_hid), lambda i: (0, 0)),
            pl.BlockSpec((1, d_hid), lambda i: (0, 0)),
            pl.BlockSpec((d_hid, d_out), lambda i: (0, 0)),
            pl.BlockSpec((1, d_out), lambda i: (0, 0)),
        ],
        out_specs=pl.BlockSpec((block_m, d_out), lambda i: (i, 0)),
        out_shape=jax.ShapeDtypeStruct((B, d_out), jnp.float32),
    )(emb, W1, b1.reshape(1, d_hid), W2, b2.reshape(1, d_out))


def kernel(input, table, W1, b1, W2, b2):
    B, K = input.shape
    V, d = table.shape
    total = B * K
    per_w = total // NW
    nchunk = per_w // CHUNK
    idx3 = (input * 4).reshape(NW, nchunk, CHUNK)
    table_pad = _tpose(table.T)
    table_lin = table_pad.reshape(4 * V, d)
    rows = _sc_gather(idx3, table_lin, total, d)
    emb = rows.reshape(B, K * d)
    return _mlp(emb, W1, b1, W2, b2)
